# baseline (speedup 1.0000x reference)
import jax
import jax.numpy as jnp
import numpy as np
from jax.experimental import pallas as pl
from jax.experimental.pallas import tpu as pltpu
from jax.sharding import Mesh, PartitionSpec as P

CH = 256
M_BLK = 256


def _body(xi_ref, gii_ref, xj_ref, gji_ref, w1_ref, b1_ref, w2_ref, b2_ref,
          out_ref, yi_ref, yj_ref):
    i = pl.program_id(0)

    @pl.when(i == 0)
    def _prologue():
        yi = jnp.dot(xi_ref[...].astype(jnp.bfloat16),
                     w1_ref[...].astype(jnp.bfloat16),
                     preferred_element_type=jnp.float32) + b1_ref[...]
        yi_ref[...] = yi.astype(jnp.bfloat16)
        yj = jnp.dot(xj_ref[...].astype(jnp.bfloat16),
                     w2_ref[...].astype(jnp.bfloat16),
                     preferred_element_type=jnp.float32) + b2_ref[...]
        yj_ref[...] = yj.astype(jnp.bfloat16)

    acc = jnp.dot(gii_ref[...].astype(jnp.bfloat16), yi_ref[...],
                  preferred_element_type=jnp.float32)
    acc = acc + jnp.dot(gji_ref[...].astype(jnp.bfloat16), yj_ref[...],
                        preferred_element_type=jnp.float32)
    out_ref[...] = jnp.maximum(acc, 0.0)


def _fused(xi, Gi2i, xj, Gj2i, W1, b1, W2, b2):
    rows = Gi2i.shape[0]
    k_i = xi.shape[0]
    k_j = xj.shape[0]
    grid = (rows // M_BLK,)
    return pl.pallas_call(
        _body,
        grid=grid,
        in_specs=[
            pl.BlockSpec((k_i, CH), lambda i: (0, 0)),
            pl.BlockSpec((M_BLK, k_i), lambda i: (i, 0)),
            pl.BlockSpec((k_j, CH), lambda i: (0, 0)),
            pl.BlockSpec((M_BLK, k_j), lambda i: (i, 0)),
            pl.BlockSpec((CH, CH), lambda i: (0, 0)),
            pl.BlockSpec((1, CH), lambda i: (0, 0)),
            pl.BlockSpec((CH, CH), lambda i: (0, 0)),
            pl.BlockSpec((1, CH), lambda i: (0, 0)),
        ],
        out_specs=pl.BlockSpec((M_BLK, CH), lambda i: (i, 0)),
        out_shape=jax.ShapeDtypeStruct((rows, CH), jnp.float32),
        scratch_shapes=[
            pltpu.VMEM((k_i, CH), jnp.bfloat16),
            pltpu.VMEM((k_j, CH), jnp.bfloat16),
        ],
    )(xi, Gi2i, xj, Gj2i, W1, b1, W2, b2)


def kernel(xi, Gi2i, xj, Gj2i, W1, b1, W2, b2):
    n_i = Gi2i.shape[0]
    devices = jax.devices()
    n_shards = 2 if (len(devices) >= 2 and (n_i // 2) % M_BLK == 0) else 1
    mesh = Mesh(np.array(devices[:n_shards]), ("x",))
    fn = jax.shard_map(
        _fused,
        mesh=mesh,
        in_specs=(P(None, None), P("x", None), P(None, None), P("x", None),
                  P(None, None), P(None, None), P(None, None), P(None, None)),
        out_specs=P("x", None),
        check_vma=False,
    )
    return fn(xi, Gi2i, xj, Gj2i, W1, b1.reshape(1, CH), W2,
              b2.reshape(1, CH))


# speedup vs baseline: 9.0908x; 9.0908x over previous
import jax
import jax.numpy as jnp
from jax.experimental import pallas as pl
from jax.experimental.pallas import tpu as pltpu

N_I_ = 4096
N_J_ = 8192
CH = 256
M_BLK = 384


def _body(xi_ref, gii_ref, xj_ref, gji_ref, w1_ref, b1_ref, w2_ref, b2_ref,
          out_ref, yi_ref, yj_ref):
    i = pl.program_id(0)

    @pl.when(i == 0)
    def _prologue():
        yi = jnp.dot(xi_ref[...].astype(jnp.bfloat16),
                     w1_ref[...].astype(jnp.bfloat16),
                     preferred_element_type=jnp.float32) + b1_ref[...]
        yi_ref[...] = yi.astype(jnp.bfloat16)
        yj = jnp.dot(xj_ref[...].astype(jnp.bfloat16),
                     w2_ref[...].astype(jnp.bfloat16),
                     preferred_element_type=jnp.float32) + b2_ref[...]
        yj_ref[...] = yj.astype(jnp.bfloat16)

    acc = jnp.dot(gii_ref[...].astype(jnp.bfloat16), yi_ref[...],
                  preferred_element_type=jnp.float32)
    acc = acc + jnp.dot(gji_ref[...].astype(jnp.bfloat16), yj_ref[...],
                        preferred_element_type=jnp.float32)
    out_ref[...] = jnp.maximum(acc, 0.0)


def kernel(xi, Gi2i, xj, Gj2i, W1, b1, W2, b2):
    n_i = Gi2i.shape[0]
    n_j = xj.shape[0]
    grid = (pl.cdiv(n_i, M_BLK),)
    return pl.pallas_call(
        _body,
        grid=grid,
        in_specs=[
            pl.BlockSpec((n_i, CH), lambda i: (0, 0)),
            pl.BlockSpec((M_BLK, n_i), lambda i: (i, 0)),
            pl.BlockSpec((n_j, CH), lambda i: (0, 0)),
            pl.BlockSpec((M_BLK, n_j), lambda i: (i, 0)),
            pl.BlockSpec((CH, CH), lambda i: (0, 0)),
            pl.BlockSpec((1, CH), lambda i: (0, 0)),
            pl.BlockSpec((CH, CH), lambda i: (0, 0)),
            pl.BlockSpec((1, CH), lambda i: (0, 0)),
        ],
        out_specs=pl.BlockSpec((M_BLK, CH), lambda i: (i, 0)),
        out_shape=jax.ShapeDtypeStruct((n_i, CH), jnp.float32),
        scratch_shapes=[
            pltpu.VMEM((n_i, CH), jnp.bfloat16),
            pltpu.VMEM((n_j, CH), jnp.bfloat16),
        ],
    )(xi, Gi2i, xj, Gj2i, W1, b1.reshape(1, CH), W2, b2.reshape(1, CH))


# speedup vs baseline: 9.1769x; 1.0095x over previous
import jax
import jax.numpy as jnp
from jax.experimental import pallas as pl
from jax.experimental.pallas import tpu as pltpu

N_I_ = 4096
N_J_ = 8192
CH = 256
M_BLK = 256


def _body(xi_ref, gii_ref, xj_ref, gji_ref, w1_ref, b1_ref, w2_ref, b2_ref,
          out_ref, yi_ref, yj_ref):
    i = pl.program_id(0)

    @pl.when(i == 0)
    def _prologue():
        yi = jnp.dot(xi_ref[...].astype(jnp.bfloat16),
                     w1_ref[...].astype(jnp.bfloat16),
                     preferred_element_type=jnp.float32) + b1_ref[...]
        yi_ref[...] = yi.astype(jnp.bfloat16)
        yj = jnp.dot(xj_ref[...].astype(jnp.bfloat16),
                     w2_ref[...].astype(jnp.bfloat16),
                     preferred_element_type=jnp.float32) + b2_ref[...]
        yj_ref[...] = yj.astype(jnp.bfloat16)

    acc = jnp.dot(gii_ref[...].astype(jnp.bfloat16), yi_ref[...],
                  preferred_element_type=jnp.float32)
    acc = acc + jnp.dot(gji_ref[...].astype(jnp.bfloat16), yj_ref[...],
                        preferred_element_type=jnp.float32)
    out_ref[...] = jnp.maximum(acc, 0.0)


def kernel(xi, Gi2i, xj, Gj2i, W1, b1, W2, b2):
    n_i = Gi2i.shape[0]
    n_j = xj.shape[0]
    grid = (n_i // M_BLK,)
    return pl.pallas_call(
        _body,
        grid=grid,
        in_specs=[
            pl.BlockSpec((n_i, CH), lambda i: (0, 0)),
            pl.BlockSpec((M_BLK, n_i), lambda i: (i, 0)),
            pl.BlockSpec((n_j, CH), lambda i: (0, 0)),
            pl.BlockSpec((M_BLK, n_j), lambda i: (i, 0)),
            pl.BlockSpec((CH, CH), lambda i: (0, 0)),
            pl.BlockSpec((1, CH), lambda i: (0, 0)),
            pl.BlockSpec((CH, CH), lambda i: (0, 0)),
            pl.BlockSpec((1, CH), lambda i: (0, 0)),
        ],
        out_specs=pl.BlockSpec((M_BLK, CH), lambda i: (i, 0)),
        out_shape=jax.ShapeDtypeStruct((n_i, CH), jnp.float32),
        scratch_shapes=[
            pltpu.VMEM((n_i, CH), jnp.bfloat16),
            pltpu.VMEM((n_j, CH), jnp.bfloat16),
        ],
    )(xi, Gi2i, xj, Gj2i, W1, b1.reshape(1, CH), W2, b2.reshape(1, CH))
